# baseline (speedup 1.0000x reference)
import functools

import jax
import jax.numpy as jnp
from jax import lax
from jax.experimental import pallas as pl
from jax.experimental.pallas import tpu as pltpu
from jax.experimental.pallas import tpu_sc as plsc

_B = 16384
_K = 32
_NW = 32
_BPW = _B // _NW
_M = 1000000
_CPW = 31248
_CW = 512
_NCH = 62
_LCAP = 768
_MCAP = 64
_ROWS_OUT = _B + 2 * _MCAP


def _extract_body(users_hbm, items_hbm, negs_hbm, pt_hbm, qt_hbm,
                  ru_hbm, ri_hbm, rn_hbm,
                  bufp0, bufp1, bufq0, bufq1, idxbuf,
                  lv0, lb0, lv1, lb1, lv2, lb2,
                  vch, bch0, bch1, bch2, bch3, bch4, bch5,
                  st0, st1, st2, st3, st4, st5,
                  semp0, semp1, semq0, semq1,
                  sems0, sems1, sems2, sems3, sems4, sems5):
    cid = lax.axis_index("c")
    sid = lax.axis_index("s")
    wid = sid * 2 + cid
    lo = wid * _CPW
    hi = jnp.where(wid == _NW - 1, _M, lo + _CPW)
    iota = lax.iota(jnp.int32, 16)
    trash = _B + iota

    bufp = (bufp0, bufp1)
    bufq = (bufq0, bufq1)
    semp = (semp0, semp1)
    semq = (semq0, semq1)
    sts = ((st0, st1), (st2, st3), (st4, st5))
    bchs = ((bch0, bch1), (bch2, bch3), (bch4, bch5))
    sems = ((sems0, sems1), (sems2, sems3), (sems4, sems5))
    lists = ((users_hbm, lv0, lb0), (items_hbm, lv1, lb1), (negs_hbm, lv2, lb2))
    outs = (ru_hbm, ri_hbm, rn_hbm)

    cnts = []
    for (src, lv, lb) in lists:
        def piece(p, off):
            pltpu.sync_copy(src.at[pl.ds(p * _CW, _CW)], idxbuf)

            def scan(v, off):
                vals = idxbuf[pl.ds(v * 16, 16)]
                m = (vals >= lo) & (vals < hi)
                plsc.store_compressed(lv.at[pl.ds(off, 16)], vals, mask=m)
                bids = p * _CW + v * 16 + iota
                plsc.store_compressed(lb.at[pl.ds(off, 16)], bids, mask=m)
                pc = plsc.all_reduce_population_count(m)
                return jnp.minimum(off + pc[0], _LCAP)

            return lax.fori_loop(0, _CW // 16, scan, off)

        cnts.append(lax.fori_loop(0, _B // _CW, piece, jnp.int32(0)))

    for t in range(3):
        for s in range(2):
            for v in range(_MCAP // 16):
                bchs[t][s][pl.ds(v * 16, 16)] = _B + (t * 2 + s) * 0 + v * 16 + iota
    prime_sc = []
    for t in range(3):
        for s in range(2):
            prime_sc.append(pltpu.async_copy(
                sts[t][s], outs[t].at[bchs[t][s]], sems[t][s]))

    def issue_streams(c, slot):
        bs = pl.multiple_of(jnp.minimum(lo + c * _CW, _M - _CW), 16)
        hp = pltpu.async_copy(pt_hbm.at[:, pl.ds(bs, _CW)], bufp[slot], semp[slot])
        hq = pltpu.async_copy(qt_hbm.at[:, pl.ds(bs, _CW)], bufq[slot], semq[slot])
        return hp, hq

    pend = [issue_streams(jnp.int32(0), 0), issue_streams(jnp.int32(1), 1)]
    for h in prime_sc:
        pass

    def process(c, slot, do_issue):
        bs = pl.multiple_of(jnp.minimum(lo + c * _CW, _M - _CW), 16)
        c_lo = lo + c * _CW
        c_hi = jnp.minimum(c_lo + _CW, hi)
        pltpu.make_async_copy(pt_hbm.at[:, pl.ds(bs, _CW)], bufp[slot],
                              semp[slot]).wait()
        pltpu.make_async_copy(qt_hbm.at[:, pl.ds(bs, _CW)], bufq[slot],
                              semq[slot]).wait()
        for t in range(3):
            lv = lists[t][1]
            lb = lists[t][2]
            cnt = cnts[t]
            buf = bufp[slot] if t == 0 else bufq[slot]
            pltpu.make_async_copy(outs[t].at[pl.ds(0, _MCAP)], sts[t][s_lot := slot],
                                  sems[t][slot]).wait()
            for v in range(_MCAP // 16):
                bchs[t][slot][pl.ds(v * 16, 16)] = _B + v * 16 + iota

            def rescan(v, moff):
                vals = lv[pl.ds(v * 16, 16)]
                bids = lb[pl.ds(v * 16, 16)]
                pos = v * 16 + iota
                m = (pos < cnt) & (vals >= c_lo) & (vals < c_hi)
                plsc.store_compressed(vch.at[pl.ds(moff, 16)], vals, mask=m)
                plsc.store_compressed(bchs[t][slot].at[pl.ds(moff, 16)], bids,
                                      mask=m)
                pc = plsc.all_reduce_population_count(m)
                return jnp.minimum(moff + pc[0], _MCAP)

            m = lax.fori_loop(0, _LCAP // 16, rescan, jnp.int32(0))

            def extract(j, carry):
                vv = vch[pl.ds(j, 16)]
                col = vv[0] - bs
                colv = jnp.broadcast_to(col, (16,))
                a = plsc.load_gather(buf, [iota, colv])
                b = plsc.load_gather(buf, [iota + 16, colv])
                sts[t][slot][j, pl.ds(0, 16)] = a
                sts[t][slot][j, pl.ds(16, 16)] = b
                return carry

            lax.fori_loop(0, m, extract, 0)
            pltpu.async_copy(sts[t][slot], outs[t].at[bchs[t][slot]],
                             sems[t][slot])
        if do_issue:
            issue_streams(c + 2, slot)

    def gbody(g, carry):
        process(g * 2, 0, True)
        process(g * 2 + 1, 1, True)
        return carry

    lax.fori_loop(0, (_NCH - 2) // 2, gbody, 0)
    process(jnp.int32(_NCH - 2), 0, False)
    process(jnp.int32(_NCH - 1), 1, False)
    for t in range(3):
        for s in range(2):
            pltpu.make_async_copy(outs[t].at[pl.ds(0, _MCAP)], sts[t][s],
                                  sems[t][s]).wait()


def _dot_body(ru_hbm, ri_hbm, rn_hbm, pos_hbm, neg_hbm,
              rows_u, rows_i, rows_n, out_p, out_n, sem):
    cid = lax.axis_index("c")
    sid = lax.axis_index("s")
    wid = sid * 2 + cid
    base = wid * _BPW
    cp = pltpu.async_copy(ru_hbm.at[pl.ds(base, _BPW)], rows_u, sem)
    ci = pltpu.async_copy(ri_hbm.at[pl.ds(base, _BPW)], rows_i, sem)
    cn = pltpu.async_copy(rn_hbm.at[pl.ds(base, _BPW)], rows_n, sem)
    cp.wait(); ci.wait(); cn.wait()

    lane = lax.iota(jnp.int32, 16)
    zeros = jnp.zeros((16,), jnp.float32)

    def group(g, carry):
        rows = g * 16 + lane
        acc_p = zeros
        acc_n = zeros
        for k in range(_K):
            colk = jnp.full((16,), k, jnp.int32)
            u = plsc.load_gather(rows_u, [rows, colk])
            i = plsc.load_gather(rows_i, [rows, colk])
            n = plsc.load_gather(rows_n, [rows, colk])
            acc_p = acc_p + u * i
            acc_n = acc_n + u * n
        out_p[pl.ds(g * 16, 16)] = acc_p
        out_n[pl.ds(g * 16, 16)] = acc_n
        return carry

    lax.fori_loop(0, _BPW // 16, group, 0)
    pltpu.sync_copy(out_p, pos_hbm.at[pl.ds(base, _BPW)])
    pltpu.sync_copy(out_n, neg_hbm.at[pl.ds(base, _BPW)])


_PARAMS = dict(
    compiler_params=pltpu.CompilerParams(
        needs_layout_passes=False, use_tc_tiling_on_sc=False),
)


@jax.jit
def _bpr(users, items, negs, Pt, Qt):
    mesh = plsc.VectorSubcoreMesh(core_axis_name="c", subcore_axis_name="s")
    f32 = jnp.float32
    i32 = jnp.int32
    extract = functools.partial(
        pl.kernel, mesh=mesh, **_PARAMS,
        out_type=tuple(jax.ShapeDtypeStruct((_ROWS_OUT, _K), f32)
                       for _ in range(3)),
        scratch_types=[
            pltpu.VMEM((_K, _CW), f32), pltpu.VMEM((_K, _CW), f32),
            pltpu.VMEM((_K, _CW), f32), pltpu.VMEM((_K, _CW), f32),
            pltpu.VMEM((_CW,), i32),
            pltpu.VMEM((_LCAP + 16,), i32), pltpu.VMEM((_LCAP + 16,), i32),
            pltpu.VMEM((_LCAP + 16,), i32), pltpu.VMEM((_LCAP + 16,), i32),
            pltpu.VMEM((_LCAP + 16,), i32), pltpu.VMEM((_LCAP + 16,), i32),
            pltpu.VMEM((_MCAP + 16,), i32),
            pltpu.VMEM((_MCAP,), i32), pltpu.VMEM((_MCAP,), i32),
            pltpu.VMEM((_MCAP,), i32), pltpu.VMEM((_MCAP,), i32),
            pltpu.VMEM((_MCAP,), i32), pltpu.VMEM((_MCAP,), i32),
            pltpu.VMEM((_MCAP, _K), f32), pltpu.VMEM((_MCAP, _K), f32),
            pltpu.VMEM((_MCAP, _K), f32), pltpu.VMEM((_MCAP, _K), f32),
            pltpu.VMEM((_MCAP, _K), f32), pltpu.VMEM((_MCAP, _K), f32),
            pltpu.SemaphoreType.DMA, pltpu.SemaphoreType.DMA,
            pltpu.SemaphoreType.DMA, pltpu.SemaphoreType.DMA,
            pltpu.SemaphoreType.DMA, pltpu.SemaphoreType.DMA,
            pltpu.SemaphoreType.DMA, pltpu.SemaphoreType.DMA,
            pltpu.SemaphoreType.DMA, pltpu.SemaphoreType.DMA,
        ],
    )(_extract_body)
    ru, ri, rn = extract(users, items, negs, Pt, Qt)
    dot = functools.partial(
        pl.kernel, mesh=mesh, **_PARAMS,
        out_type=(jax.ShapeDtypeStruct((_B,), f32),
                  jax.ShapeDtypeStruct((_B,), f32)),
        scratch_types=[
            pltpu.VMEM((_BPW, _K), f32), pltpu.VMEM((_BPW, _K), f32),
            pltpu.VMEM((_BPW, _K), f32),
            pltpu.VMEM((_BPW,), f32), pltpu.VMEM((_BPW,), f32),
            pltpu.SemaphoreType.DMA,
        ],
    )(_dot_body)
    return dot(ru, ri, rn)


def kernel(users, items, neg_items, P, Q):
    pos, neg = _bpr(users.astype(jnp.int32), items.astype(jnp.int32),
                    neg_items.astype(jnp.int32), P.T, Q.T)
    return (pos, neg)


# speedup vs baseline: 1.0237x; 1.0237x over previous
import functools

import jax
import jax.numpy as jnp
from jax import lax
from jax.experimental import pallas as pl
from jax.experimental.pallas import tpu as pltpu
from jax.experimental.pallas import tpu_sc as plsc

_B = 16384
_K = 32
_NW = 32
_BPW = _B // _NW
_M = 1000000
_CPW = 31248
_CW = 512
_NCH = 62
_LCAP = 768
_MCAP = 64
_ROWS_OUT = _B + _NW * 6 * _MCAP


def _extract_body(users_hbm, items_hbm, negs_hbm, pt_hbm, qt_hbm,
                  ru_hbm, ri_hbm, rn_hbm,
                  bufp0, bufp1, bufq0, bufq1, idxbuf,
                  lv0, lb0, lv1, lb1, lv2, lb2,
                  vch, bch0, bch1, bch2, bch3, bch4, bch5,
                  st0, st1, st2, st3, st4, st5,
                  semp0, semp1, semq0, semq1,
                  sems0, sems1, sems2, sems3, sems4, sems5):
    cid = lax.axis_index("c")
    sid = lax.axis_index("s")
    wid = sid * 2 + cid
    lo = wid * _CPW
    hi = jnp.where(wid == _NW - 1, _M, lo + _CPW)
    iota = lax.iota(jnp.int32, 16)
    trash = _B + iota

    bufp = (bufp0, bufp1)
    bufq = (bufq0, bufq1)
    semp = (semp0, semp1)
    semq = (semq0, semq1)
    sts = ((st0, st1), (st2, st3), (st4, st5))
    bchs = ((bch0, bch1), (bch2, bch3), (bch4, bch5))
    sems = ((sems0, sems1), (sems2, sems3), (sems4, sems5))
    lists = ((users_hbm, lv0, lb0), (items_hbm, lv1, lb1), (negs_hbm, lv2, lb2))
    outs = (ru_hbm, ri_hbm, rn_hbm)

    cnts = []
    for (src, lv, lb) in lists:
        def piece(p, off):
            pltpu.sync_copy(src.at[pl.ds(p * _CW, _CW)], idxbuf)

            def scan(v, off):
                vals = idxbuf[pl.ds(v * 16, 16)]
                m = (vals >= lo) & (vals < hi)
                plsc.store_compressed(lv.at[pl.ds(off, 16)], vals, mask=m)
                bids = p * _CW + v * 16 + iota
                plsc.store_compressed(lb.at[pl.ds(off, 16)], bids, mask=m)
                pc = plsc.all_reduce_population_count(m)
                return jnp.minimum(off + pc[0], _LCAP)

            return lax.fori_loop(0, _CW // 16, scan, off)

        cnts.append(lax.fori_loop(0, _B // _CW, piece, jnp.int32(0)))

    def trash_base(t, s):
        return _B + (wid * 6 + t * 2 + s) * _MCAP

    for t in range(3):
        for s in range(2):
            for v in range(_MCAP // 16):
                bchs[t][s][pl.ds(v * 16, 16)] = trash_base(t, s) + v * 16 + iota
    prime_sc = []
    for t in range(3):
        for s in range(2):
            prime_sc.append(pltpu.async_copy(
                sts[t][s], outs[t].at[bchs[t][s]], sems[t][s]))

    def issue_streams(c, slot):
        bs = pl.multiple_of(jnp.minimum(lo + c * _CW, _M - _CW), 16)
        hp = pltpu.async_copy(pt_hbm.at[:, pl.ds(bs, _CW)], bufp[slot], semp[slot])
        hq = pltpu.async_copy(qt_hbm.at[:, pl.ds(bs, _CW)], bufq[slot], semq[slot])
        return hp, hq

    pend = [issue_streams(jnp.int32(0), 0), issue_streams(jnp.int32(1), 1)]
    for h in prime_sc:
        pass

    def process(c, slot, do_issue):
        bs = pl.multiple_of(jnp.minimum(lo + c * _CW, _M - _CW), 16)
        c_lo = lo + c * _CW
        c_hi = jnp.minimum(c_lo + _CW, hi)
        pltpu.make_async_copy(pt_hbm.at[:, pl.ds(bs, _CW)], bufp[slot],
                              semp[slot]).wait()
        pltpu.make_async_copy(qt_hbm.at[:, pl.ds(bs, _CW)], bufq[slot],
                              semq[slot]).wait()
        for t in range(3):
            lv = lists[t][1]
            lb = lists[t][2]
            cnt = cnts[t]
            buf = bufp[slot] if t == 0 else bufq[slot]
            pltpu.make_async_copy(outs[t].at[pl.ds(0, _MCAP)], sts[t][s_lot := slot],
                                  sems[t][slot]).wait()
            for v in range(_MCAP // 16):
                bchs[t][slot][pl.ds(v * 16, 16)] = trash_base(t, slot) + v * 16 + iota

            def rescan(v, moff):
                vals = lv[pl.ds(v * 16, 16)]
                bids = lb[pl.ds(v * 16, 16)]
                pos = v * 16 + iota
                m = (pos < cnt) & (vals >= c_lo) & (vals < c_hi)
                plsc.store_compressed(vch.at[pl.ds(moff, 16)], vals, mask=m)
                plsc.store_compressed(bchs[t][slot].at[pl.ds(moff, 16)], bids,
                                      mask=m)
                pc = plsc.all_reduce_population_count(m)
                return jnp.minimum(moff + pc[0], _MCAP)

            m = lax.fori_loop(0, _LCAP // 16, rescan, jnp.int32(0))

            def extract(j, carry):
                vv = vch[pl.ds(j, 16)]
                col = vv[0] - bs
                colv = jnp.broadcast_to(col, (16,))
                a = plsc.load_gather(buf, [iota, colv])
                b = plsc.load_gather(buf, [iota + 16, colv])
                sts[t][slot][j, pl.ds(0, 16)] = a
                sts[t][slot][j, pl.ds(16, 16)] = b
                return carry

            lax.fori_loop(0, m, extract, 0)
            pltpu.async_copy(sts[t][slot], outs[t].at[bchs[t][slot]],
                             sems[t][slot])
        if do_issue:
            issue_streams(c + 2, slot)

    def gbody(g, carry):
        process(g * 2, 0, True)
        process(g * 2 + 1, 1, True)
        return carry

    lax.fori_loop(0, (_NCH - 2) // 2, gbody, 0)
    process(jnp.int32(_NCH - 2), 0, False)
    process(jnp.int32(_NCH - 1), 1, False)
    for t in range(3):
        for s in range(2):
            pltpu.make_async_copy(outs[t].at[pl.ds(0, _MCAP)], sts[t][s],
                                  sems[t][s]).wait()


def _dot_body(ru_hbm, ri_hbm, rn_hbm, pos_hbm, neg_hbm,
              rows_u, rows_i, rows_n, out_p, out_n, sem):
    cid = lax.axis_index("c")
    sid = lax.axis_index("s")
    wid = sid * 2 + cid
    base = wid * _BPW
    cp = pltpu.async_copy(ru_hbm.at[pl.ds(base, _BPW)], rows_u, sem)
    ci = pltpu.async_copy(ri_hbm.at[pl.ds(base, _BPW)], rows_i, sem)
    cn = pltpu.async_copy(rn_hbm.at[pl.ds(base, _BPW)], rows_n, sem)
    cp.wait(); ci.wait(); cn.wait()

    lane = lax.iota(jnp.int32, 16)
    zeros = jnp.zeros((16,), jnp.float32)

    def group(g, carry):
        rows = g * 16 + lane
        acc_p = zeros
        acc_n = zeros
        for k in range(_K):
            colk = jnp.full((16,), k, jnp.int32)
            u = plsc.load_gather(rows_u, [rows, colk])
            i = plsc.load_gather(rows_i, [rows, colk])
            n = plsc.load_gather(rows_n, [rows, colk])
            acc_p = acc_p + u * i
            acc_n = acc_n + u * n
        out_p[pl.ds(g * 16, 16)] = acc_p
        out_n[pl.ds(g * 16, 16)] = acc_n
        return carry

    lax.fori_loop(0, _BPW // 16, group, 0)
    pltpu.sync_copy(out_p, pos_hbm.at[pl.ds(base, _BPW)])
    pltpu.sync_copy(out_n, neg_hbm.at[pl.ds(base, _BPW)])


_PARAMS = dict(
    compiler_params=pltpu.CompilerParams(
        needs_layout_passes=False, use_tc_tiling_on_sc=False),
)


@jax.jit
def _bpr(users, items, negs, Pt, Qt):
    mesh = plsc.VectorSubcoreMesh(core_axis_name="c", subcore_axis_name="s")
    f32 = jnp.float32
    i32 = jnp.int32
    extract = functools.partial(
        pl.kernel, mesh=mesh, **_PARAMS,
        out_type=tuple(jax.ShapeDtypeStruct((_ROWS_OUT, _K), f32)
                       for _ in range(3)),
        scratch_types=[
            pltpu.VMEM((_K, _CW), f32), pltpu.VMEM((_K, _CW), f32),
            pltpu.VMEM((_K, _CW), f32), pltpu.VMEM((_K, _CW), f32),
            pltpu.VMEM((_CW,), i32),
            pltpu.VMEM((_LCAP + 16,), i32), pltpu.VMEM((_LCAP + 16,), i32),
            pltpu.VMEM((_LCAP + 16,), i32), pltpu.VMEM((_LCAP + 16,), i32),
            pltpu.VMEM((_LCAP + 16,), i32), pltpu.VMEM((_LCAP + 16,), i32),
            pltpu.VMEM((_MCAP + 16,), i32),
            pltpu.VMEM((_MCAP,), i32), pltpu.VMEM((_MCAP,), i32),
            pltpu.VMEM((_MCAP,), i32), pltpu.VMEM((_MCAP,), i32),
            pltpu.VMEM((_MCAP,), i32), pltpu.VMEM((_MCAP,), i32),
            pltpu.VMEM((_MCAP, _K), f32), pltpu.VMEM((_MCAP, _K), f32),
            pltpu.VMEM((_MCAP, _K), f32), pltpu.VMEM((_MCAP, _K), f32),
            pltpu.VMEM((_MCAP, _K), f32), pltpu.VMEM((_MCAP, _K), f32),
            pltpu.SemaphoreType.DMA, pltpu.SemaphoreType.DMA,
            pltpu.SemaphoreType.DMA, pltpu.SemaphoreType.DMA,
            pltpu.SemaphoreType.DMA, pltpu.SemaphoreType.DMA,
            pltpu.SemaphoreType.DMA, pltpu.SemaphoreType.DMA,
            pltpu.SemaphoreType.DMA, pltpu.SemaphoreType.DMA,
        ],
    )(_extract_body)
    ru, ri, rn = extract(users, items, negs, Pt, Qt)
    dot = functools.partial(
        pl.kernel, mesh=mesh, **_PARAMS,
        out_type=(jax.ShapeDtypeStruct((_B,), f32),
                  jax.ShapeDtypeStruct((_B,), f32)),
        scratch_types=[
            pltpu.VMEM((_BPW, _K), f32), pltpu.VMEM((_BPW, _K), f32),
            pltpu.VMEM((_BPW, _K), f32),
            pltpu.VMEM((_BPW,), f32), pltpu.VMEM((_BPW,), f32),
            pltpu.SemaphoreType.DMA,
        ],
    )(_dot_body)
    return dot(ru, ri, rn)


def kernel(users, items, neg_items, P, Q):
    pos, neg = _bpr(users.astype(jnp.int32), items.astype(jnp.int32),
                    neg_items.astype(jnp.int32), P.T, Q.T)
    return (pos, neg)


# speedup vs baseline: 1.0244x; 1.0007x over previous
import functools

import jax
import jax.numpy as jnp
from jax import lax
from jax.experimental import pallas as pl
from jax.experimental.pallas import tpu as pltpu
from jax.experimental.pallas import tpu_sc as plsc

_B = 16384
_K = 32
_NW = 32
_BPW = _B // _NW
_M = 1000000
_CPW = 31248
_CW = 512
_NCH = 62
_LCAP = 768
_MCAP = 64
_ROWS_OUT = _B + _NW * 6 * _MCAP


def _extract_body(users_hbm, items_hbm, negs_hbm, pt_hbm, qt_hbm,
                  ru_hbm, ri_hbm, rn_hbm,
                  bufp0, bufp1, bufq0, bufq1, idxbuf,
                  lv0, lb0, lv1, lb1, lv2, lb2,
                  vch, bch0, bch1, bch2, bch3, bch4, bch5,
                  st0, st1, st2, st3, st4, st5,
                  semp0, semp1, semq0, semq1,
                  sems0, sems1, sems2, sems3, sems4, sems5):
    cid = lax.axis_index("c")
    sid = lax.axis_index("s")
    wid = sid * 2 + cid
    lo = wid * _CPW
    hi = jnp.where(wid == _NW - 1, _M, lo + _CPW)
    iota = lax.iota(jnp.int32, 16)
    trash = _B + iota

    bufp = (bufp0, bufp1)
    bufq = (bufq0, bufq1)
    semp = (semp0, semp1)
    semq = (semq0, semq1)
    sts = ((st0, st1), (st2, st3), (st4, st5))
    bchs = ((bch0, bch1), (bch2, bch3), (bch4, bch5))
    sems = ((sems0, sems1), (sems2, sems3), (sems4, sems5))
    lists = ((users_hbm, lv0, lb0), (items_hbm, lv1, lb1), (negs_hbm, lv2, lb2))
    outs = (ru_hbm, ri_hbm, rn_hbm)

    cnts = []
    for (src, lv, lb) in lists:
        def piece(p, off):
            pltpu.sync_copy(src.at[pl.ds(p * _CW, _CW)], idxbuf)

            def scan(v, off):
                vals = idxbuf[pl.ds(v * 16, 16)]
                m = (vals >= lo) & (vals < hi)
                plsc.store_compressed(lv.at[pl.ds(off, 16)], vals, mask=m)
                bids = p * _CW + v * 16 + iota
                plsc.store_compressed(lb.at[pl.ds(off, 16)], bids, mask=m)
                pc = plsc.all_reduce_population_count(m)
                return jnp.minimum(off + pc[0], _LCAP)

            return lax.fori_loop(0, _CW // 16, scan, off)

        cnts.append(lax.fori_loop(0, _B // _CW, piece, jnp.int32(0)))

    def trash_base(t, s):
        return _B + (wid * 6 + t * 2 + s) * _MCAP

    for t in range(3):
        for s in range(2):
            for v in range(_MCAP // 16):
                bchs[t][s][pl.ds(v * 16, 16)] = trash_base(t, s) + v * 16 + iota
    prime_sc = []
    for t in range(3):
        for s in range(2):
            prime_sc.append(pltpu.async_copy(
                sts[t][s], outs[t].at[bchs[t][s]], sems[t][s]))

    def issue_streams(c, slot):
        bs = pl.multiple_of(jnp.minimum(lo + c * _CW, _M - _CW), 16)
        hp = pltpu.async_copy(pt_hbm.at[:, pl.ds(bs, _CW)], bufp[slot], semp[slot])
        hq = pltpu.async_copy(qt_hbm.at[:, pl.ds(bs, _CW)], bufq[slot], semq[slot])
        return hp, hq

    pend = [issue_streams(jnp.int32(0), 0), issue_streams(jnp.int32(1), 1)]
    for h in prime_sc:
        pass

    def process(c, slot, do_issue):
        bs = pl.multiple_of(jnp.minimum(lo + c * _CW, _M - _CW), 16)
        c_lo = lo + c * _CW
        c_hi = jnp.minimum(c_lo + _CW, hi)
        pltpu.make_async_copy(pt_hbm.at[:, pl.ds(bs, _CW)], bufp[slot],
                              semp[slot]).wait()
        pltpu.make_async_copy(qt_hbm.at[:, pl.ds(bs, _CW)], bufq[slot],
                              semq[slot]).wait()
        for t in range(3):
            lv = lists[t][1]
            lb = lists[t][2]
            cnt = cnts[t]
            buf = bufp[slot] if t == 0 else bufq[slot]
            pltpu.make_async_copy(sts[t][slot], outs[t].at[bchs[t][slot]],
                                  sems[t][slot]).wait()
            for v in range(_MCAP // 16):
                bchs[t][slot][pl.ds(v * 16, 16)] = trash_base(t, slot) + v * 16 + iota

            def rescan(v, moff):
                vals = lv[pl.ds(v * 16, 16)]
                bids = lb[pl.ds(v * 16, 16)]
                pos = v * 16 + iota
                m = (pos < cnt) & (vals >= c_lo) & (vals < c_hi)
                plsc.store_compressed(vch.at[pl.ds(moff, 16)], vals, mask=m)
                plsc.store_compressed(bchs[t][slot].at[pl.ds(moff, 16)], bids,
                                      mask=m)
                pc = plsc.all_reduce_population_count(m)
                return jnp.minimum(moff + pc[0], _MCAP)

            m = lax.fori_loop(0, _LCAP // 16, rescan, jnp.int32(0))

            def extract(j, carry):
                vv = vch[pl.ds(j, 16)]
                col = vv[0] - bs
                colv = jnp.broadcast_to(col, (16,))
                a = plsc.load_gather(buf, [iota, colv])
                b = plsc.load_gather(buf, [iota + 16, colv])
                sts[t][slot][j, pl.ds(0, 16)] = a
                sts[t][slot][j, pl.ds(16, 16)] = b
                return carry

            lax.fori_loop(0, m, extract, 0)
            pltpu.async_copy(sts[t][slot], outs[t].at[bchs[t][slot]],
                             sems[t][slot])
        if do_issue:
            issue_streams(c + 2, slot)

    def gbody(g, carry):
        process(g * 2, 0, True)
        process(g * 2 + 1, 1, True)
        return carry

    lax.fori_loop(0, (_NCH - 2) // 2, gbody, 0)
    process(jnp.int32(_NCH - 2), 0, False)
    process(jnp.int32(_NCH - 1), 1, False)
    for t in range(3):
        for s in range(2):
            pltpu.make_async_copy(sts[t][s], outs[t].at[bchs[t][s]],
                                  sems[t][s]).wait()


def _dot_body(ru_hbm, ri_hbm, rn_hbm, pos_hbm, neg_hbm,
              rows_u, rows_i, rows_n, out_p, out_n, sem):
    cid = lax.axis_index("c")
    sid = lax.axis_index("s")
    wid = sid * 2 + cid
    base = wid * _BPW
    cp = pltpu.async_copy(ru_hbm.at[pl.ds(base, _BPW)], rows_u, sem)
    ci = pltpu.async_copy(ri_hbm.at[pl.ds(base, _BPW)], rows_i, sem)
    cn = pltpu.async_copy(rn_hbm.at[pl.ds(base, _BPW)], rows_n, sem)
    cp.wait(); ci.wait(); cn.wait()

    lane = lax.iota(jnp.int32, 16)
    zeros = jnp.zeros((16,), jnp.float32)

    def group(g, carry):
        rows = g * 16 + lane
        acc_p = zeros
        acc_n = zeros
        for k in range(_K):
            colk = jnp.full((16,), k, jnp.int32)
            u = plsc.load_gather(rows_u, [rows, colk])
            i = plsc.load_gather(rows_i, [rows, colk])
            n = plsc.load_gather(rows_n, [rows, colk])
            acc_p = acc_p + u * i
            acc_n = acc_n + u * n
        out_p[pl.ds(g * 16, 16)] = acc_p
        out_n[pl.ds(g * 16, 16)] = acc_n
        return carry

    lax.fori_loop(0, _BPW // 16, group, 0)
    pltpu.sync_copy(out_p, pos_hbm.at[pl.ds(base, _BPW)])
    pltpu.sync_copy(out_n, neg_hbm.at[pl.ds(base, _BPW)])


_PARAMS = dict(
    compiler_params=pltpu.CompilerParams(
        needs_layout_passes=False, use_tc_tiling_on_sc=False),
)


@jax.jit
def _bpr(users, items, negs, Pt, Qt):
    mesh = plsc.VectorSubcoreMesh(core_axis_name="c", subcore_axis_name="s")
    f32 = jnp.float32
    i32 = jnp.int32
    extract = functools.partial(
        pl.kernel, mesh=mesh, **_PARAMS,
        out_type=tuple(jax.ShapeDtypeStruct((_ROWS_OUT, _K), f32)
                       for _ in range(3)),
        scratch_types=[
            pltpu.VMEM((_K, _CW), f32), pltpu.VMEM((_K, _CW), f32),
            pltpu.VMEM((_K, _CW), f32), pltpu.VMEM((_K, _CW), f32),
            pltpu.VMEM((_CW,), i32),
            pltpu.VMEM((_LCAP + 16,), i32), pltpu.VMEM((_LCAP + 16,), i32),
            pltpu.VMEM((_LCAP + 16,), i32), pltpu.VMEM((_LCAP + 16,), i32),
            pltpu.VMEM((_LCAP + 16,), i32), pltpu.VMEM((_LCAP + 16,), i32),
            pltpu.VMEM((_MCAP + 16,), i32),
            pltpu.VMEM((_MCAP,), i32), pltpu.VMEM((_MCAP,), i32),
            pltpu.VMEM((_MCAP,), i32), pltpu.VMEM((_MCAP,), i32),
            pltpu.VMEM((_MCAP,), i32), pltpu.VMEM((_MCAP,), i32),
            pltpu.VMEM((_MCAP, _K), f32), pltpu.VMEM((_MCAP, _K), f32),
            pltpu.VMEM((_MCAP, _K), f32), pltpu.VMEM((_MCAP, _K), f32),
            pltpu.VMEM((_MCAP, _K), f32), pltpu.VMEM((_MCAP, _K), f32),
            pltpu.SemaphoreType.DMA, pltpu.SemaphoreType.DMA,
            pltpu.SemaphoreType.DMA, pltpu.SemaphoreType.DMA,
            pltpu.SemaphoreType.DMA, pltpu.SemaphoreType.DMA,
            pltpu.SemaphoreType.DMA, pltpu.SemaphoreType.DMA,
            pltpu.SemaphoreType.DMA, pltpu.SemaphoreType.DMA,
        ],
    )(_extract_body)
    ru, ri, rn = extract(users, items, negs, Pt, Qt)
    dot = functools.partial(
        pl.kernel, mesh=mesh, **_PARAMS,
        out_type=(jax.ShapeDtypeStruct((_B,), f32),
                  jax.ShapeDtypeStruct((_B,), f32)),
        scratch_types=[
            pltpu.VMEM((_BPW, _K), f32), pltpu.VMEM((_BPW, _K), f32),
            pltpu.VMEM((_BPW, _K), f32),
            pltpu.VMEM((_BPW,), f32), pltpu.VMEM((_BPW,), f32),
            pltpu.SemaphoreType.DMA,
        ],
    )(_dot_body)
    return dot(ru, ri, rn)


def kernel(users, items, neg_items, P, Q):
    pos, neg = _bpr(users.astype(jnp.int32), items.astype(jnp.int32),
                    neg_items.astype(jnp.int32), P.T, Q.T)
    return (pos, neg)


# speedup vs baseline: 1.0255x; 1.0011x over previous
import functools

import jax
import jax.numpy as jnp
from jax import lax
from jax.experimental import pallas as pl
from jax.experimental.pallas import tpu as pltpu
from jax.experimental.pallas import tpu_sc as plsc

_B = 16384
_K = 32
_NW = 32
_BPW = _B // _NW
_M = 1000000
_CPW = 31248
_CW = 512
_NCH = 62
_LCAP = 768
_MCAP = 128
_ROWS_OUT = _B + _NW * 6 * _MCAP


def _extract_body(users_hbm, items_hbm, negs_hbm, pt_hbm, qt_hbm,
                  ru_hbm, ri_hbm, rn_hbm,
                  bufp0, bufp1, bufq0, bufq1, idxbuf,
                  lv0, lb0, lv1, lb1, lv2, lb2,
                  vch, bch0, bch1, bch2, bch3, bch4, bch5,
                  st0, st1, st2, st3, st4, st5,
                  semp0, semp1, semq0, semq1,
                  sems0, sems1, sems2, sems3, sems4, sems5):
    cid = lax.axis_index("c")
    sid = lax.axis_index("s")
    wid = sid * 2 + cid
    lo = wid * _CPW
    hi = jnp.where(wid == _NW - 1, _M, lo + _CPW)
    iota = lax.iota(jnp.int32, 16)
    trash = _B + iota

    bufp = (bufp0, bufp1)
    bufq = (bufq0, bufq1)
    semp = (semp0, semp1)
    semq = (semq0, semq1)
    sts = ((st0, st1), (st2, st3), (st4, st5))
    bchs = ((bch0, bch1), (bch2, bch3), (bch4, bch5))
    sems = ((sems0, sems1), (sems2, sems3), (sems4, sems5))
    lists = ((users_hbm, lv0, lb0), (items_hbm, lv1, lb1), (negs_hbm, lv2, lb2))
    outs = (ru_hbm, ri_hbm, rn_hbm)

    cnts = []
    for (src, lv, lb) in lists:
        def piece(p, off):
            pltpu.sync_copy(src.at[pl.ds(p * _CW, _CW)], idxbuf)

            def scan(v, off):
                vals = idxbuf[pl.ds(v * 16, 16)]
                m = (vals >= lo) & (vals < hi)
                plsc.store_compressed(lv.at[pl.ds(off, 16)], vals, mask=m)
                bids = p * _CW + v * 16 + iota
                plsc.store_compressed(lb.at[pl.ds(off, 16)], bids, mask=m)
                pc = plsc.all_reduce_population_count(m)
                return jnp.minimum(off + pc[0], _LCAP)

            return lax.fori_loop(0, _CW // 16, scan, off)

        cnts.append(lax.fori_loop(0, _B // _CW, piece, jnp.int32(0)))

    def trash_base(t, s):
        return _B + (wid * 6 + t * 2 + s) * _MCAP

    for t in range(3):
        for s in range(2):
            for v in range(_MCAP // 16):
                bchs[t][s][0, pl.ds(v * 16, 16)] = trash_base(t, s) + v * 16 + iota
    prime_sc = []
    for t in range(3):
        for s in range(2):
            prime_sc.append(pltpu.async_copy(
                sts[t][s], outs[t].at[bchs[t][s].at[0]], sems[t][s]))

    def issue_streams(c, slot):
        bs = pl.multiple_of(jnp.minimum(lo + c * _CW, _M - _CW), 16)
        hp = pltpu.async_copy(pt_hbm.at[:, pl.ds(bs, _CW)], bufp[slot], semp[slot])
        hq = pltpu.async_copy(qt_hbm.at[:, pl.ds(bs, _CW)], bufq[slot], semq[slot])
        return hp, hq

    pend = [issue_streams(jnp.int32(0), 0), issue_streams(jnp.int32(1), 1)]
    for h in prime_sc:
        pass

    def process(c, slot, do_issue):
        bs = pl.multiple_of(jnp.minimum(lo + c * _CW, _M - _CW), 16)
        c_lo = lo + c * _CW
        c_hi = jnp.minimum(c_lo + _CW, hi)
        pltpu.make_async_copy(pt_hbm.at[:, pl.ds(bs, _CW)], bufp[slot],
                              semp[slot]).wait()
        pltpu.make_async_copy(qt_hbm.at[:, pl.ds(bs, _CW)], bufq[slot],
                              semq[slot]).wait()
        for t in range(3):
            lv = lists[t][1]
            lb = lists[t][2]
            cnt = cnts[t]
            buf = bufp[slot] if t == 0 else bufq[slot]
            pltpu.make_async_copy(sts[t][slot], outs[t].at[bchs[t][slot].at[0]],
                                  sems[t][slot]).wait()
            for v in range(_MCAP // 16):
                bchs[t][slot][0, pl.ds(v * 16, 16)] = trash_base(t, slot) + v * 16 + iota

            def rescan(v, moff):
                vals = lv[pl.ds(v * 16, 16)]
                bids = lb[pl.ds(v * 16, 16)]
                pos = v * 16 + iota
                m = (pos < cnt) & (vals >= c_lo) & (vals < c_hi)
                plsc.store_compressed(vch.at[pl.ds(moff, 16)], vals, mask=m)
                plsc.store_compressed(bchs[t][slot].at[0, pl.ds(moff, 16)], bids,
                                      mask=m)
                pc = plsc.all_reduce_population_count(m)
                return jnp.minimum(moff + pc[0], _MCAP)

            m = lax.fori_loop(0, _LCAP // 16, rescan, jnp.int32(0))

            def extract(j, carry):
                vv = vch[pl.ds(j, 16)]
                col = vv[0] - bs
                colv = jnp.broadcast_to(col, (16,))
                a = plsc.load_gather(buf, [iota, colv])
                b = plsc.load_gather(buf, [iota + 16, colv])
                sts[t][slot][j, pl.ds(0, 16)] = a
                sts[t][slot][j, pl.ds(16, 16)] = b
                return carry

            lax.fori_loop(0, m, extract, 0)
            pltpu.async_copy(sts[t][slot], outs[t].at[bchs[t][slot].at[0]],
                             sems[t][slot])
        if do_issue:
            issue_streams(c + 2, slot)

    def gbody(g, carry):
        process(g * 2, 0, True)
        process(g * 2 + 1, 1, True)
        return carry

    lax.fori_loop(0, (_NCH - 2) // 2, gbody, 0)
    process(jnp.int32(_NCH - 2), 0, False)
    process(jnp.int32(_NCH - 1), 1, False)
    for t in range(3):
        for s in range(2):
            pltpu.make_async_copy(sts[t][s], outs[t].at[bchs[t][s].at[0]],
                                  sems[t][s]).wait()


def _dot_body(ru_hbm, ri_hbm, rn_hbm, pos_hbm, neg_hbm,
              rows_u, rows_i, rows_n, out_p, out_n, sem):
    cid = lax.axis_index("c")
    sid = lax.axis_index("s")
    wid = sid * 2 + cid
    base = wid * _BPW
    cp = pltpu.async_copy(ru_hbm.at[pl.ds(base, _BPW)], rows_u, sem)
    ci = pltpu.async_copy(ri_hbm.at[pl.ds(base, _BPW)], rows_i, sem)
    cn = pltpu.async_copy(rn_hbm.at[pl.ds(base, _BPW)], rows_n, sem)
    cp.wait(); ci.wait(); cn.wait()

    lane = lax.iota(jnp.int32, 16)
    zeros = jnp.zeros((16,), jnp.float32)

    def group(g, carry):
        rows = g * 16 + lane
        acc_p = zeros
        acc_n = zeros
        for k in range(_K):
            colk = jnp.full((16,), k, jnp.int32)
            u = plsc.load_gather(rows_u, [rows, colk])
            i = plsc.load_gather(rows_i, [rows, colk])
            n = plsc.load_gather(rows_n, [rows, colk])
            acc_p = acc_p + u * i
            acc_n = acc_n + u * n
        out_p[pl.ds(g * 16, 16)] = acc_p
        out_n[pl.ds(g * 16, 16)] = acc_n
        return carry

    lax.fori_loop(0, _BPW // 16, group, 0)
    pltpu.sync_copy(out_p, pos_hbm.at[pl.ds(base, _BPW)])
    pltpu.sync_copy(out_n, neg_hbm.at[pl.ds(base, _BPW)])


_PARAMS = dict(
    compiler_params=pltpu.CompilerParams(
        needs_layout_passes=False, use_tc_tiling_on_sc=False),
)


@jax.jit
def _bpr(users, items, negs, Pt, Qt):
    mesh = plsc.VectorSubcoreMesh(core_axis_name="c", subcore_axis_name="s")
    f32 = jnp.float32
    i32 = jnp.int32
    extract = functools.partial(
        pl.kernel, mesh=mesh, **_PARAMS,
        out_type=tuple(jax.ShapeDtypeStruct((_ROWS_OUT, _K), f32)
                       for _ in range(3)),
        scratch_types=[
            pltpu.VMEM((_K, _CW), f32), pltpu.VMEM((_K, _CW), f32),
            pltpu.VMEM((_K, _CW), f32), pltpu.VMEM((_K, _CW), f32),
            pltpu.VMEM((_CW,), i32),
            pltpu.VMEM((_LCAP + 16,), i32), pltpu.VMEM((_LCAP + 16,), i32),
            pltpu.VMEM((_LCAP + 16,), i32), pltpu.VMEM((_LCAP + 16,), i32),
            pltpu.VMEM((_LCAP + 16,), i32), pltpu.VMEM((_LCAP + 16,), i32),
            pltpu.VMEM((_MCAP + 16,), i32),
            pltpu.VMEM((1, _MCAP), i32), pltpu.VMEM((1, _MCAP), i32),
            pltpu.VMEM((1, _MCAP), i32), pltpu.VMEM((1, _MCAP), i32),
            pltpu.VMEM((1, _MCAP), i32), pltpu.VMEM((1, _MCAP), i32),
            pltpu.VMEM((_MCAP, _K), f32), pltpu.VMEM((_MCAP, _K), f32),
            pltpu.VMEM((_MCAP, _K), f32), pltpu.VMEM((_MCAP, _K), f32),
            pltpu.VMEM((_MCAP, _K), f32), pltpu.VMEM((_MCAP, _K), f32),
            pltpu.SemaphoreType.DMA, pltpu.SemaphoreType.DMA,
            pltpu.SemaphoreType.DMA, pltpu.SemaphoreType.DMA,
            pltpu.SemaphoreType.DMA, pltpu.SemaphoreType.DMA,
            pltpu.SemaphoreType.DMA, pltpu.SemaphoreType.DMA,
            pltpu.SemaphoreType.DMA, pltpu.SemaphoreType.DMA,
        ],
    )(_extract_body)
    ru, ri, rn = extract(users, items, negs, Pt, Qt)
    dot = functools.partial(
        pl.kernel, mesh=mesh, **_PARAMS,
        out_type=(jax.ShapeDtypeStruct((_B,), f32),
                  jax.ShapeDtypeStruct((_B,), f32)),
        scratch_types=[
            pltpu.VMEM((_BPW, _K), f32), pltpu.VMEM((_BPW, _K), f32),
            pltpu.VMEM((_BPW, _K), f32),
            pltpu.VMEM((_BPW,), f32), pltpu.VMEM((_BPW,), f32),
            pltpu.SemaphoreType.DMA,
        ],
    )(_dot_body)
    return dot(ru, ri, rn)


def kernel(users, items, neg_items, P, Q):
    pos, neg = _bpr(users.astype(jnp.int32), items.astype(jnp.int32),
                    neg_items.astype(jnp.int32), P.T, Q.T)
    return (pos, neg)


# speedup vs baseline: 18.1577x; 17.7062x over previous
import functools

import jax
import jax.numpy as jnp
from jax import lax
from jax.experimental import pallas as pl
from jax.experimental.pallas import tpu as pltpu
from jax.experimental.pallas import tpu_sc as plsc

_B = 16384
_K = 32
_NW = 32
_BPW = _B // _NW
_M = 1000000
_MT = 999936
_CPW = 31232
_CW = 512
_NCH = 62
_LCAP = 768
_MCAP = 48
_RW = 128
_ROWS_OUT = _B + _NW * 7 * _MCAP


def _extract_body(users_hbm, items_hbm, negs_hbm, pt_hbm, qt_hbm,
                  ptail_hbm, qtail_hbm,
                  ru_hbm, ri_hbm, rn_hbm,
                  bufp0, bufp1, bufq0, bufq1, idxbuf, tailp, tailq,
                  lv0, lb0, lv1, lb1, lv2, lb2,
                  vch, bch0, bch1, bch2, bch3, bch4, bch5,
                  st0, st1, st2, st3, st4, st5,
                  semp0, semp1, semq0, semq1,
                  sems0, sems1, sems2, sems3, sems4, sems5):
    cid = lax.axis_index("c")
    sid = lax.axis_index("s")
    wid = sid * 2 + cid
    lo = wid * _CPW
    hi = jnp.where(wid == _NW - 1, _M, lo + _CPW)
    iota = lax.iota(jnp.int32, 16)

    bufp = (bufp0, bufp1)
    bufq = (bufq0, bufq1)
    semp = (semp0, semp1)
    semq = (semq0, semq1)
    sts = ((st0, st1), (st2, st3), (st4, st5))
    bchs = ((bch0, bch1), (bch2, bch3), (bch4, bch5))
    sems = ((sems0, sems1), (sems2, sems3), (sems4, sems5))
    lists = ((users_hbm, lv0, lb0), (items_hbm, lv1, lb1), (negs_hbm, lv2, lb2))
    outs = (ru_hbm, ri_hbm, rn_hbm)

    pltpu.sync_copy(ptail_hbm, tailp)
    pltpu.sync_copy(qtail_hbm, tailq)

    cnts = []
    for (src, lv, lb) in lists:
        def piece(p, off):
            pltpu.sync_copy(src.at[pl.ds(p * _CW, _CW)], idxbuf)

            def scan(v, off):
                vals = idxbuf[pl.ds(v * 16, 16)]
                m = (vals >= lo) & (vals < hi)
                plsc.store_compressed(lv.at[pl.ds(off, 16)], vals, mask=m)
                bids = p * _CW + v * 16 + iota
                plsc.store_compressed(lb.at[pl.ds(off, 16)], bids, mask=m)
                pc = plsc.all_reduce_population_count(m)
                return jnp.minimum(off + pc[0], _LCAP)

            return lax.fori_loop(0, _CW // 16, scan, off)

        cnts.append(lax.fori_loop(0, _B // _CW, piece, jnp.int32(0)))

    def trash_base(t, s):
        return _B + (wid * 7 + t * 2 + s) * _MCAP

    for t in range(3):
        for s in range(2):
            for v in range(_MCAP // 16):
                bchs[t][s][0, pl.ds(v * 16, 16)] = trash_base(t, s) + v * 16 + iota
    for t in range(3):
        for s in range(2):
            pltpu.async_copy(sts[t][s], outs[t].at[bchs[t][s].at[0]], sems[t][s])

    def issue_streams(c, slot):
        bs = pl.multiple_of(jnp.minimum(lo + c * _CW, _MT - _CW), 128)
        pltpu.async_copy(pt_hbm.at[:, pl.ds(bs, _CW)], bufp[slot], semp[slot])
        pltpu.async_copy(qt_hbm.at[:, pl.ds(bs, _CW)], bufq[slot], semq[slot])

    issue_streams(jnp.int32(0), 0)
    issue_streams(jnp.int32(1), 1)

    def process(c, slot, do_issue):
        bs = pl.multiple_of(jnp.minimum(lo + c * _CW, _MT - _CW), 128)
        c_lo = lo + c * _CW
        c_hi = jnp.minimum(c_lo + _CW, jnp.minimum(hi, _MT))
        pltpu.make_async_copy(pt_hbm.at[:, pl.ds(bs, _CW)], bufp[slot],
                              semp[slot]).wait()
        pltpu.make_async_copy(qt_hbm.at[:, pl.ds(bs, _CW)], bufq[slot],
                              semq[slot]).wait()
        for t in range(3):
            lv = lists[t][1]
            lb = lists[t][2]
            cnt = cnts[t]
            buf = bufp[slot] if t == 0 else bufq[slot]
            pltpu.make_async_copy(sts[t][slot], outs[t].at[bchs[t][slot].at[0]],
                                  sems[t][slot]).wait()
            for v in range(_MCAP // 16):
                bchs[t][slot][0, pl.ds(v * 16, 16)] = (
                    trash_base(t, slot) + v * 16 + iota)

            def rescan(v, moff):
                vals = lv[pl.ds(v * 16, 16)]
                bids = lb[pl.ds(v * 16, 16)]
                pos = v * 16 + iota
                m = (pos < cnt) & (vals >= c_lo) & (vals < c_hi)
                plsc.store_compressed(vch.at[pl.ds(moff, 16)], vals, mask=m)
                plsc.store_compressed(bchs[t][slot].at[0, pl.ds(moff, 16)],
                                      bids, mask=m)
                pc = plsc.all_reduce_population_count(m)
                return jnp.minimum(moff + pc[0], _MCAP)

            m = lax.fori_loop(0, _LCAP // 16, rescan, jnp.int32(0))

            def extract(j, carry):
                vv = vch[pl.ds(j, 16)]
                col = vv[0] - bs
                colv = jnp.broadcast_to(col, (16,))
                a = plsc.load_gather(buf, [iota, colv])
                b = plsc.load_gather(buf, [iota + 16, colv])
                sts[t][slot][j, pl.ds(0, 16)] = a
                sts[t][slot][j, pl.ds(16, 16)] = b
                return carry

            lax.fori_loop(0, m, extract, 0)
            pltpu.async_copy(sts[t][slot], outs[t].at[bchs[t][slot].at[0]],
                             sems[t][slot])
        if do_issue:
            issue_streams(c + 2, slot)

    def gbody(g, carry):
        process(g * 2, 0, True)
        process(g * 2 + 1, 1, True)
        return carry

    lax.fori_loop(0, (_NCH - 2) // 2, gbody, 0)
    process(jnp.int32(_NCH - 2), 0, False)
    process(jnp.int32(_NCH - 1), 1, False)

    for t in range(3):
        lv = lists[t][1]
        lb = lists[t][2]
        cnt = cnts[t]
        buf = tailp if t == 0 else tailq
        slot = 0
        pltpu.make_async_copy(sts[t][slot], outs[t].at[bchs[t][slot].at[0]],
                              sems[t][slot]).wait()
        tb = _B + (wid * 7 + 6) * _MCAP
        for v in range(_MCAP // 16):
            bchs[t][slot][0, pl.ds(v * 16, 16)] = tb + v * 16 + iota

        def rescan_t(v, moff):
            vals = lv[pl.ds(v * 16, 16)]
            bids = lb[pl.ds(v * 16, 16)]
            pos = v * 16 + iota
            m = (pos < cnt) & (vals >= _MT)
            plsc.store_compressed(vch.at[pl.ds(moff, 16)], vals, mask=m)
            plsc.store_compressed(bchs[t][slot].at[0, pl.ds(moff, 16)],
                                  bids, mask=m)
            pc = plsc.all_reduce_population_count(m)
            return jnp.minimum(moff + pc[0], _MCAP)

        m = lax.fori_loop(0, _LCAP // 16, rescan_t, jnp.int32(0))

        def extract_t(j, carry):
            vv = vch[pl.ds(j, 16)]
            col = vv[0] - _MT
            colv = jnp.broadcast_to(col, (16,))
            a = plsc.load_gather(buf, [iota, colv])
            b = plsc.load_gather(buf, [iota + 16, colv])
            sts[t][slot][j, pl.ds(0, 16)] = a
            sts[t][slot][j, pl.ds(16, 16)] = b
            return carry

        lax.fori_loop(0, m, extract_t, 0)
        pltpu.async_copy(sts[t][slot], outs[t].at[bchs[t][slot].at[0]],
                         sems[t][slot])

    for t in range(3):
        for s in range(2):
            pltpu.make_async_copy(sts[t][s], outs[t].at[bchs[t][s].at[0]],
                                  sems[t][s]).wait()


def _dot_body(ru_hbm, ri_hbm, rn_hbm, pos_hbm, neg_hbm,
              rows_u, rows_i, rows_n, out_p, out_n, sem):
    cid = lax.axis_index("c")
    sid = lax.axis_index("s")
    wid = sid * 2 + cid
    lane = lax.iota(jnp.int32, 16)
    zeros = jnp.zeros((16,), jnp.float32)

    for half in range(2):
        base = wid * _BPW + half * (_BPW // 2)
        cu = pltpu.async_copy(ru_hbm.at[pl.ds(base, _BPW // 2)], rows_u, sem)
        ci = pltpu.async_copy(ri_hbm.at[pl.ds(base, _BPW // 2)], rows_i, sem)
        cn = pltpu.async_copy(rn_hbm.at[pl.ds(base, _BPW // 2)], rows_n, sem)
        cu.wait(); ci.wait(); cn.wait()

        def group(g, carry):
            rows = g * 16 + lane
            acc_p = zeros
            acc_n = zeros
            for k in range(_K):
                colk = jnp.full((16,), k, jnp.int32)
                u = plsc.load_gather(rows_u, [rows, colk])
                i = plsc.load_gather(rows_i, [rows, colk])
                n = plsc.load_gather(rows_n, [rows, colk])
                acc_p = acc_p + u * i
                acc_n = acc_n + u * n
            out_p[pl.ds(half * (_BPW // 2) + g * 16, 16)] = acc_p
            out_n[pl.ds(half * (_BPW // 2) + g * 16, 16)] = acc_n
            return carry

        lax.fori_loop(0, _BPW // 32, group, 0)

    obase = wid * _BPW
    pltpu.sync_copy(out_p, pos_hbm.at[pl.ds(obase, _BPW)])
    pltpu.sync_copy(out_n, neg_hbm.at[pl.ds(obase, _BPW)])


_PARAMS = dict(
    compiler_params=pltpu.CompilerParams(
        needs_layout_passes=False, use_tc_tiling_on_sc=True),
)


@jax.jit
def _bpr(users, items, negs, Pt, Qt, Ptail, Qtail):
    mesh = plsc.VectorSubcoreMesh(core_axis_name="c", subcore_axis_name="s")
    f32 = jnp.float32
    i32 = jnp.int32
    extract = functools.partial(
        pl.kernel, mesh=mesh, **_PARAMS,
        out_type=tuple(jax.ShapeDtypeStruct((_ROWS_OUT, _RW), f32)
                       for _ in range(3)),
        scratch_types=[
            pltpu.VMEM((_K, _CW), f32), pltpu.VMEM((_K, _CW), f32),
            pltpu.VMEM((_K, _CW), f32), pltpu.VMEM((_K, _CW), f32),
            pltpu.VMEM((_CW,), i32),
            pltpu.VMEM((_K, 64), f32), pltpu.VMEM((_K, 64), f32),
            pltpu.VMEM((_LCAP + 16,), i32), pltpu.VMEM((_LCAP + 16,), i32),
            pltpu.VMEM((_LCAP + 16,), i32), pltpu.VMEM((_LCAP + 16,), i32),
            pltpu.VMEM((_LCAP + 16,), i32), pltpu.VMEM((_LCAP + 16,), i32),
            pltpu.VMEM((_MCAP + 16,), i32),
            pltpu.VMEM((1, _MCAP), i32), pltpu.VMEM((1, _MCAP), i32),
            pltpu.VMEM((1, _MCAP), i32), pltpu.VMEM((1, _MCAP), i32),
            pltpu.VMEM((1, _MCAP), i32), pltpu.VMEM((1, _MCAP), i32),
            pltpu.VMEM((_MCAP, _RW), f32), pltpu.VMEM((_MCAP, _RW), f32),
            pltpu.VMEM((_MCAP, _RW), f32), pltpu.VMEM((_MCAP, _RW), f32),
            pltpu.VMEM((_MCAP, _RW), f32), pltpu.VMEM((_MCAP, _RW), f32),
            pltpu.SemaphoreType.DMA, pltpu.SemaphoreType.DMA,
            pltpu.SemaphoreType.DMA, pltpu.SemaphoreType.DMA,
            pltpu.SemaphoreType.DMA, pltpu.SemaphoreType.DMA,
            pltpu.SemaphoreType.DMA, pltpu.SemaphoreType.DMA,
            pltpu.SemaphoreType.DMA, pltpu.SemaphoreType.DMA,
        ],
    )(_extract_body)
    ru, ri, rn = extract(users, items, negs, Pt, Qt, Ptail, Qtail)
    dot = functools.partial(
        pl.kernel, mesh=mesh, **_PARAMS,
        out_type=(jax.ShapeDtypeStruct((_B,), f32),
                  jax.ShapeDtypeStruct((_B,), f32)),
        scratch_types=[
            pltpu.VMEM((_BPW // 2, _RW), f32), pltpu.VMEM((_BPW // 2, _RW), f32),
            pltpu.VMEM((_BPW // 2, _RW), f32),
            pltpu.VMEM((_BPW,), f32), pltpu.VMEM((_BPW,), f32),
            pltpu.SemaphoreType.DMA,
        ],
    )(_dot_body)
    return dot(ru, ri, rn)


def kernel(users, items, neg_items, P, Q):
    Pt = P.T
    Qt = Q.T
    pos, neg = _bpr(users.astype(jnp.int32), items.astype(jnp.int32),
                    neg_items.astype(jnp.int32), Pt, Qt,
                    Pt[:, _MT:], Qt[:, _MT:])
    return (pos, neg)


# speedup vs baseline: 18.6345x; 1.0263x over previous
import functools

import jax
import jax.numpy as jnp
from jax import lax
from jax.experimental import pallas as pl
from jax.experimental.pallas import tpu as pltpu
from jax.experimental.pallas import tpu_sc as plsc

_B = 16384
_K = 32
_NW = 32
_BPW = _B // _NW
_M = 1000000
_MT = 999936
_CPW = 31232
_CW = 512
_NCH = 62
_LCAP = 768
_MCAP = 48
_RW = 128
_ROWS_OUT = _B + _NW * 7 * _MCAP


def _extract_body(users_hbm, items_hbm, negs_hbm, pt_hbm, qt_hbm,
                  ptail_hbm, qtail_hbm,
                  ru_hbm, ri_hbm, rn_hbm,
                  bufp0, bufp1, bufq0, bufq1, idxbuf, tailp, tailq,
                  lv0, lb0, lv1, lb1, lv2, lb2,
                  vch, bch0, bch1, bch2, bch3, bch4, bch5,
                  st0, st1, st2, st3, st4, st5,
                  semp0, semp1, semq0, semq1,
                  sems0, sems1, sems2, sems3, sems4, sems5):
    cid = lax.axis_index("c")
    sid = lax.axis_index("s")
    wid = sid * 2 + cid
    lo = wid * _CPW
    hi = jnp.where(wid == _NW - 1, _M, lo + _CPW)
    iota = lax.iota(jnp.int32, 16)

    bufp = (bufp0, bufp1)
    bufq = (bufq0, bufq1)
    semp = (semp0, semp1)
    semq = (semq0, semq1)
    sts = ((st0, st1), (st2, st3), (st4, st5))
    bchs = ((bch0, bch1), (bch2, bch3), (bch4, bch5))
    sems = ((sems0, sems1), (sems2, sems3), (sems4, sems5))
    lists = ((users_hbm, lv0, lb0), (items_hbm, lv1, lb1), (negs_hbm, lv2, lb2))
    outs = (ru_hbm, ri_hbm, rn_hbm)

    pltpu.sync_copy(ptail_hbm, tailp)
    pltpu.sync_copy(qtail_hbm, tailq)

    cnts = []
    for (src, lv, lb) in lists:
        def piece(p, off):
            pltpu.sync_copy(src.at[pl.ds(p * _CW, _CW)], idxbuf)

            def scan(v, off):
                vals = idxbuf[pl.ds(v * 16, 16)]
                m = (vals >= lo) & (vals < hi)
                plsc.store_compressed(lv.at[pl.ds(off, 16)], vals, mask=m)
                bids = p * _CW + v * 16 + iota
                plsc.store_compressed(lb.at[pl.ds(off, 16)], bids, mask=m)
                pc = plsc.all_reduce_population_count(m)
                return jnp.minimum(off + pc[0], _LCAP)

            return lax.fori_loop(0, _CW // 16, scan, off)

        cnts.append(lax.fori_loop(0, _B // _CW, piece, jnp.int32(0)))

    def trash_base(t, s):
        return _B + (wid * 7 + t * 2 + s) * _MCAP

    for t in range(3):
        for s in range(2):
            for v in range(_MCAP // 16):
                bchs[t][s][0, pl.ds(v * 16, 16)] = trash_base(t, s) + v * 16 + iota
    for t in range(3):
        for s in range(2):
            pltpu.async_copy(sts[t][s], outs[t].at[bchs[t][s].at[0]], sems[t][s])

    def issue_streams(c, slot):
        bs = pl.multiple_of(jnp.minimum(lo + c * _CW, _MT - _CW), 128)
        pltpu.async_copy(pt_hbm.at[:, pl.ds(bs, _CW)], bufp[slot], semp[slot])
        pltpu.async_copy(qt_hbm.at[:, pl.ds(bs, _CW)], bufq[slot], semq[slot])

    issue_streams(jnp.int32(0), 0)
    issue_streams(jnp.int32(1), 1)

    def process(c, slot, do_issue):
        bs = pl.multiple_of(jnp.minimum(lo + c * _CW, _MT - _CW), 128)
        c_lo = lo + c * _CW
        c_hi = jnp.minimum(c_lo + _CW, jnp.minimum(hi, _MT))
        pltpu.make_async_copy(pt_hbm.at[:, pl.ds(bs, _CW)], bufp[slot],
                              semp[slot]).wait()
        pltpu.make_async_copy(qt_hbm.at[:, pl.ds(bs, _CW)], bufq[slot],
                              semq[slot]).wait()
        for t in range(3):
            lv = lists[t][1]
            lb = lists[t][2]
            cnt = cnts[t]
            buf = bufp[slot] if t == 0 else bufq[slot]
            pltpu.make_async_copy(sts[t][slot], outs[t].at[bchs[t][slot].at[0]],
                                  sems[t][slot]).wait()
            for v in range(_MCAP // 16):
                bchs[t][slot][0, pl.ds(v * 16, 16)] = (
                    trash_base(t, slot) + v * 16 + iota)

            def rescan(v, moff):
                vals = lv[pl.ds(v * 16, 16)]
                bids = lb[pl.ds(v * 16, 16)]
                pos = v * 16 + iota
                m = (pos < cnt) & (vals >= c_lo) & (vals < c_hi)
                plsc.store_compressed(vch.at[pl.ds(moff, 16)], vals, mask=m)
                plsc.store_compressed(bchs[t][slot].at[0, pl.ds(moff, 16)],
                                      bids, mask=m)
                pc = plsc.all_reduce_population_count(m)
                return jnp.minimum(moff + pc[0], _MCAP)

            m = lax.fori_loop(0, (cnt + 15) // 16, rescan, jnp.int32(0))

            def extract(j, carry):
                vv = vch[pl.ds(j, 16)]
                col = vv[0] - bs
                colv = jnp.broadcast_to(col, (16,))
                a = plsc.load_gather(buf, [iota, colv])
                b = plsc.load_gather(buf, [iota + 16, colv])
                sts[t][slot][j, pl.ds(0, 16)] = a
                sts[t][slot][j, pl.ds(16, 16)] = b
                return carry

            lax.fori_loop(0, m, extract, 0)
            pltpu.async_copy(sts[t][slot], outs[t].at[bchs[t][slot].at[0]],
                             sems[t][slot])
        if do_issue:
            issue_streams(c + 2, slot)

    def gbody(g, carry):
        process(g * 2, 0, True)
        process(g * 2 + 1, 1, True)
        return carry

    lax.fori_loop(0, (_NCH - 2) // 2, gbody, 0)
    process(jnp.int32(_NCH - 2), 0, False)
    process(jnp.int32(_NCH - 1), 1, False)

    for t in range(3):
        lv = lists[t][1]
        lb = lists[t][2]
        cnt = cnts[t]
        buf = tailp if t == 0 else tailq
        slot = 0
        pltpu.make_async_copy(sts[t][slot], outs[t].at[bchs[t][slot].at[0]],
                              sems[t][slot]).wait()
        tb = _B + (wid * 7 + 6) * _MCAP
        for v in range(_MCAP // 16):
            bchs[t][slot][0, pl.ds(v * 16, 16)] = tb + v * 16 + iota

        def rescan_t(v, moff):
            vals = lv[pl.ds(v * 16, 16)]
            bids = lb[pl.ds(v * 16, 16)]
            pos = v * 16 + iota
            m = (pos < cnt) & (vals >= _MT)
            plsc.store_compressed(vch.at[pl.ds(moff, 16)], vals, mask=m)
            plsc.store_compressed(bchs[t][slot].at[0, pl.ds(moff, 16)],
                                  bids, mask=m)
            pc = plsc.all_reduce_population_count(m)
            return jnp.minimum(moff + pc[0], _MCAP)

        m = lax.fori_loop(0, (cnt + 15) // 16, rescan_t, jnp.int32(0))

        def extract_t(j, carry):
            vv = vch[pl.ds(j, 16)]
            col = vv[0] - _MT
            colv = jnp.broadcast_to(col, (16,))
            a = plsc.load_gather(buf, [iota, colv])
            b = plsc.load_gather(buf, [iota + 16, colv])
            sts[t][slot][j, pl.ds(0, 16)] = a
            sts[t][slot][j, pl.ds(16, 16)] = b
            return carry

        lax.fori_loop(0, m, extract_t, 0)
        pltpu.async_copy(sts[t][slot], outs[t].at[bchs[t][slot].at[0]],
                         sems[t][slot])

    for t in range(3):
        for s in range(2):
            pltpu.make_async_copy(sts[t][s], outs[t].at[bchs[t][s].at[0]],
                                  sems[t][s]).wait()


def _dot_body(ru_hbm, ri_hbm, rn_hbm, pos_hbm, neg_hbm,
              rows_u, rows_i, rows_n, out_p, out_n, sem):
    cid = lax.axis_index("c")
    sid = lax.axis_index("s")
    wid = sid * 2 + cid
    lane = lax.iota(jnp.int32, 16)
    zeros = jnp.zeros((16,), jnp.float32)

    for half in range(2):
        base = wid * _BPW + half * (_BPW // 2)
        cu = pltpu.async_copy(ru_hbm.at[pl.ds(base, _BPW // 2)], rows_u, sem)
        ci = pltpu.async_copy(ri_hbm.at[pl.ds(base, _BPW // 2)], rows_i, sem)
        cn = pltpu.async_copy(rn_hbm.at[pl.ds(base, _BPW // 2)], rows_n, sem)
        cu.wait(); ci.wait(); cn.wait()

        def group(g, carry):
            rows = g * 16 + lane
            acc_p = zeros
            acc_n = zeros
            for k in range(_K):
                colk = jnp.full((16,), k, jnp.int32)
                u = plsc.load_gather(rows_u, [rows, colk])
                i = plsc.load_gather(rows_i, [rows, colk])
                n = plsc.load_gather(rows_n, [rows, colk])
                acc_p = acc_p + u * i
                acc_n = acc_n + u * n
            out_p[pl.ds(half * (_BPW // 2) + g * 16, 16)] = acc_p
            out_n[pl.ds(half * (_BPW // 2) + g * 16, 16)] = acc_n
            return carry

        lax.fori_loop(0, _BPW // 32, group, 0)

    obase = wid * _BPW
    pltpu.sync_copy(out_p, pos_hbm.at[pl.ds(obase, _BPW)])
    pltpu.sync_copy(out_n, neg_hbm.at[pl.ds(obase, _BPW)])


_PARAMS = dict(
    compiler_params=pltpu.CompilerParams(
        needs_layout_passes=False, use_tc_tiling_on_sc=True),
)


@jax.jit
def _bpr(users, items, negs, Pt, Qt, Ptail, Qtail):
    mesh = plsc.VectorSubcoreMesh(core_axis_name="c", subcore_axis_name="s")
    f32 = jnp.float32
    i32 = jnp.int32
    extract = functools.partial(
        pl.kernel, mesh=mesh, **_PARAMS,
        out_type=tuple(jax.ShapeDtypeStruct((_ROWS_OUT, _RW), f32)
                       for _ in range(3)),
        scratch_types=[
            pltpu.VMEM((_K, _CW), f32), pltpu.VMEM((_K, _CW), f32),
            pltpu.VMEM((_K, _CW), f32), pltpu.VMEM((_K, _CW), f32),
            pltpu.VMEM((_CW,), i32),
            pltpu.VMEM((_K, 64), f32), pltpu.VMEM((_K, 64), f32),
            pltpu.VMEM((_LCAP + 16,), i32), pltpu.VMEM((_LCAP + 16,), i32),
            pltpu.VMEM((_LCAP + 16,), i32), pltpu.VMEM((_LCAP + 16,), i32),
            pltpu.VMEM((_LCAP + 16,), i32), pltpu.VMEM((_LCAP + 16,), i32),
            pltpu.VMEM((_MCAP + 16,), i32),
            pltpu.VMEM((1, _MCAP), i32), pltpu.VMEM((1, _MCAP), i32),
            pltpu.VMEM((1, _MCAP), i32), pltpu.VMEM((1, _MCAP), i32),
            pltpu.VMEM((1, _MCAP), i32), pltpu.VMEM((1, _MCAP), i32),
            pltpu.VMEM((_MCAP, _RW), f32), pltpu.VMEM((_MCAP, _RW), f32),
            pltpu.VMEM((_MCAP, _RW), f32), pltpu.VMEM((_MCAP, _RW), f32),
            pltpu.VMEM((_MCAP, _RW), f32), pltpu.VMEM((_MCAP, _RW), f32),
            pltpu.SemaphoreType.DMA, pltpu.SemaphoreType.DMA,
            pltpu.SemaphoreType.DMA, pltpu.SemaphoreType.DMA,
            pltpu.SemaphoreType.DMA, pltpu.SemaphoreType.DMA,
            pltpu.SemaphoreType.DMA, pltpu.SemaphoreType.DMA,
            pltpu.SemaphoreType.DMA, pltpu.SemaphoreType.DMA,
        ],
    )(_extract_body)
    ru, ri, rn = extract(users, items, negs, Pt, Qt, Ptail, Qtail)
    dot = functools.partial(
        pl.kernel, mesh=mesh, **_PARAMS,
        out_type=(jax.ShapeDtypeStruct((_B,), f32),
                  jax.ShapeDtypeStruct((_B,), f32)),
        scratch_types=[
            pltpu.VMEM((_BPW // 2, _RW), f32), pltpu.VMEM((_BPW // 2, _RW), f32),
            pltpu.VMEM((_BPW // 2, _RW), f32),
            pltpu.VMEM((_BPW,), f32), pltpu.VMEM((_BPW,), f32),
            pltpu.SemaphoreType.DMA,
        ],
    )(_dot_body)
    return dot(ru, ri, rn)


def kernel(users, items, neg_items, P, Q):
    Pt = P.T
    Qt = Q.T
    pos, neg = _bpr(users.astype(jnp.int32), items.astype(jnp.int32),
                    neg_items.astype(jnp.int32), Pt, Qt,
                    Pt[:, _MT:], Qt[:, _MT:])
    return (pos, neg)
